# Initial kernel scaffold; baseline (speedup 1.0000x reference)
import jax
import jax.numpy as jnp
from jax.experimental import pallas as pl


def kernel(edge_index, edge_type, bases1, comp1, root1, bias1, bases2, comp2, root2, bias2, emb_rel):
    raise NotImplementedError("write your pallas kernel here")


# speedup vs baseline: 3.3921x; 3.3921x over previous
import jax
import jax.numpy as jnp
from jax import lax
from jax.experimental import pallas as pl
from jax.experimental.pallas import tpu as pltpu
from jax.experimental.pallas import tpu_sc as plsc

N = 10000
E = 320000
R = 16
H = 100
NB = 30
HP = 128
NP = 10240
NR = N * R
NC = 2
NS = 16
NW = NC * NS
C = 1280
CB = 128
NJ = C // CB
NJP = 16
NCHUNK = E // C
TPW = (NCHUNK + NW - 1) // NW
NPT = NP // NS
CPT = 10240

_mesh = plsc.VectorSubcoreMesh(
    core_axis_name="c", subcore_axis_name="s", num_cores=NC, num_subcores=NS)


def _worker_ids():
    c = lax.axis_index("c")
    s = lax.axis_index("s")
    return c, s, c * NS + s


def _prep_body(es_ref, ed_ref, et_ref,
               cnt2_ref, idx1_ref, idx2_ref, comb_ref, dst_ref,
               src_v, dst_v, rel_v, i1_v, i2_v, cb_v, d2_v, ones_v, zb_v,
               cnt_sh):
    c, s, wid = _worker_ids()
    for k in range(CB // 16):
        ones_v[pl.ds(k * 16, 16)] = jnp.ones((16,), jnp.float32)

    def zrow(k, carry):
        zb_v[pl.ds(k * 16, 16)] = jnp.zeros((16,), jnp.float32)
        return carry

    lax.fori_loop(0, 2048 // 16, zrow, 0)
    for j in range(CPT // 2048):
        pltpu.sync_copy(zb_v, cnt_sh.at[pl.ds(s * CPT + j * 2048, 2048)])
    plsc.subcore_barrier()

    def chunk(t, carry):
        i = wid + t * NW

        @pl.when(i < NCHUNK)
        def _():
            base = i * C
            pltpu.sync_copy(es_ref.at[pl.ds(base, C)], src_v)
            pltpu.sync_copy(ed_ref.at[pl.ds(base, C)], dst_v)
            pltpu.sync_copy(et_ref.at[pl.ds(base, C)], rel_v)
            for k in range(C // 16):
                row = k // (CB // 16)
                col = (k % (CB // 16)) * 16
                sl = pl.ds(k * 16, 16)
                sv = src_v[sl]
                dv = dst_v[sl]
                rv = rel_v[sl]
                cb_v[row, pl.ds(col, 16)] = dv * R + rv
                i1_v[row, pl.ds(col, 16)] = rv * N + sv
                i2_v[row, pl.ds(col, 16)] = sv * R + rv
                d2_v[row, pl.ds(col, 16)] = dv
            for j in range(NJ):
                pltpu.sync_copy(ones_v, cnt_sh.at[cb_v.at[j]], add=True)
            pltpu.sync_copy(i1_v, idx1_ref.at[i])
            pltpu.sync_copy(i2_v, idx2_ref.at[i])
            pltpu.sync_copy(cb_v, comb_ref.at[i])
            pltpu.sync_copy(d2_v, dst_ref.at[i])
        return carry

    lax.fori_loop(0, TPW, chunk, 0)
    plsc.subcore_barrier()
    pltpu.sync_copy(cnt_sh.at[pl.ds(s * CPT, CPT)], cnt2_ref.at[c, s])


_prep = pl.kernel(
    _prep_body,
    out_type=(
        jax.ShapeDtypeStruct((NC, NS, CPT), jnp.float32),
        jax.ShapeDtypeStruct((NCHUNK, NJP, CB), jnp.int32),
        jax.ShapeDtypeStruct((NCHUNK, NJP, CB), jnp.int32),
        jax.ShapeDtypeStruct((NCHUNK, NJP, CB), jnp.int32),
        jax.ShapeDtypeStruct((NCHUNK, NJP, CB), jnp.int32),
    ),
    mesh=_mesh,
    scratch_types=[
        pltpu.VMEM((C,), jnp.int32),
        pltpu.VMEM((C,), jnp.int32),
        pltpu.VMEM((C,), jnp.int32),
        pltpu.VMEM((NJP, CB), jnp.int32),
        pltpu.VMEM((NJP, CB), jnp.int32),
        pltpu.VMEM((NJP, CB), jnp.int32),
        pltpu.VMEM((NJP, CB), jnp.int32),
        pltpu.VMEM((CB,), jnp.float32),
        pltpu.VMEM((2048,), jnp.float32),
        pltpu.VMEM_SHARED((NS * CPT,), jnp.float32),
    ],
)


def _conv_body(tab_ref, rcnt_ref, idx_ref, comb_ref, dst_ref,
               part_ref,
               i_v, c_v, d_v, n_v, rows_v, acc_sh):
    c, s, wid = _worker_ids()

    def zrow(r, carry):
        for q in range(HP // 16):
            rows_v[r, pl.ds(q * 16, 16)] = jnp.zeros((16,), jnp.float32)
        return carry

    lax.fori_loop(0, CB, zrow, 0)
    for j in range(NPT // CB):
        pltpu.sync_copy(rows_v, acc_sh.at[pl.ds(s * NPT + j * CB, CB)])
    plsc.subcore_barrier()

    def chunk(t, carry):
        i = wid + t * NW

        @pl.when(i < NCHUNK)
        def _():
            pltpu.sync_copy(idx_ref.at[i], i_v)
            pltpu.sync_copy(comb_ref.at[i], c_v)
            pltpu.sync_copy(dst_ref.at[i], d_v)
            for j in range(NJ):
                pltpu.sync_copy(rcnt_ref.at[c_v.at[j]],
                                n_v.at[pl.ds(j * CB, CB)])
            for j in range(NJ):
                pltpu.sync_copy(tab_ref.at[i_v.at[j]], rows_v)

                def scale(e, carry2):
                    nbv = n_v[pl.ds(e + j * CB, 16)]
                    nb = jnp.full((16,), nbv[0], jnp.float32)
                    for q in range(HP // 16):
                        sl = pl.ds(q * 16, 16)
                        rows_v[e, sl] = rows_v[e, sl] * nb
                    return carry2

                lax.fori_loop(0, CB, scale, 0)
                pltpu.sync_copy(rows_v, acc_sh.at[d_v.at[j]], add=True)
        return carry

    lax.fori_loop(0, TPW, chunk, 0)
    plsc.subcore_barrier()
    pltpu.sync_copy(acc_sh.at[pl.ds(s * NPT, NPT)], part_ref.at[c, s])


_conv = pl.kernel(
    _conv_body,
    out_type=jax.ShapeDtypeStruct((NC, NS, NPT, HP), jnp.float32),
    mesh=_mesh,
    scratch_types=[
        pltpu.VMEM((NJP, CB), jnp.int32),
        pltpu.VMEM((NJP, CB), jnp.int32),
        pltpu.VMEM((NJP, CB), jnp.int32),
        pltpu.VMEM((NJP * CB,), jnp.float32),
        pltpu.VMEM((CB, HP), jnp.float32),
        pltpu.VMEM_SHARED((NP, HP), jnp.float32),
    ],
)


def _rel_body(tbl_ref, et_ref, out_ref, rel_v, rows_v):
    c, s, wid = _worker_ids()

    def chunk(t, carry):
        i = wid + t * NW

        @pl.when(i < NCHUNK)
        def _():
            base = i * C
            pltpu.sync_copy(et_ref.at[pl.ds(base, C)], rel_v)
            for j in range(NJ):
                pltpu.sync_copy(tbl_ref.at[rel_v.at[pl.ds(j * CB, CB)]],
                                rows_v)
                pltpu.sync_copy(rows_v, out_ref.at[pl.ds(base + j * CB, CB)])
        return carry

    lax.fori_loop(0, TPW, chunk, 0)


_rel = pl.kernel(
    _rel_body,
    out_type=jax.ShapeDtypeStruct((E, HP), jnp.float32),
    mesh=_mesh,
    scratch_types=[
        pltpu.VMEM((C,), jnp.int32),
        pltpu.VMEM((CB, HP), jnp.float32),
    ],
)


def _rcnt_body(cnt_ref, out_ref):
    out_ref[...] = 1.0 / jnp.maximum(cnt_ref[0] + cnt_ref[1], 1.0)


def _rcnt(cnt2):
    cnt3 = cnt2.reshape(NC, NR // 128, 128)
    out = pl.pallas_call(
        _rcnt_body,
        grid=(1,),
        in_specs=[pl.BlockSpec((NC, NR // 128, 128), lambda i: (0, 0, 0))],
        out_specs=pl.BlockSpec((NR // 128, 128), lambda i: (0, 0)),
        out_shape=jax.ShapeDtypeStruct((NR // 128, 128), jnp.float32),
    )(cnt3)
    return out.reshape(NR)


_BN1 = 1000


def _w1p_body(comp_ref, bases_ref, out_ref):
    b = bases_ref[...].reshape(NB, _BN1 * H)
    m = jnp.dot(comp_ref[...], b,
                preferred_element_type=jnp.float32).reshape(R, _BN1, H)
    pad = jnp.zeros((R, _BN1, HP - H), jnp.float32)
    out_ref[...] = jnp.concatenate([m, pad], axis=2)


def _w1p(comp1, bases1):
    out = pl.pallas_call(
        _w1p_body,
        grid=(N // _BN1,),
        in_specs=[
            pl.BlockSpec((R, NB), lambda i: (0, 0)),
            pl.BlockSpec((NB, _BN1, H), lambda i: (0, i, 0)),
        ],
        out_specs=pl.BlockSpec((R, _BN1, HP), lambda i: (0, i, 0)),
        out_shape=jax.ShapeDtypeStruct((R, N, HP), jnp.float32),
    )(comp1, bases1)
    return out.reshape(R * N, HP)


def _w2prep_body(comp_ref, bases_ref, emb_ref, w2m_ref, tbl_ref):
    sel = (lax.broadcasted_iota(jnp.int32, (1, R), 1)
           == pl.program_id(0)).astype(jnp.float32)
    row = jnp.dot(sel, comp_ref[...], preferred_element_type=jnp.float32)
    v = jnp.zeros((H, H), jnp.float32)
    for b in range(NB):
        v = v + row[0, b] * bases_ref[b]
    v = jnp.concatenate([v, jnp.zeros((H, HP - H), jnp.float32)], axis=1)
    v = jnp.concatenate([v, jnp.zeros((HP - H, HP), jnp.float32)], axis=0)
    w2m_ref[...] = v[None]
    e = emb_ref[...]
    ep = jnp.concatenate(
        [jnp.zeros((1, H), jnp.float32), e[1:]], axis=0)
    ep = jnp.concatenate(
        [ep, jnp.zeros((R + 1, HP - H), jnp.float32)], axis=1)
    tbl_ref[...] = jnp.concatenate(
        [ep, jnp.zeros((32 - (R + 1), HP), jnp.float32)], axis=0)


def _w2prep(comp2, bases2, emb_rel):
    return pl.pallas_call(
        _w2prep_body,
        grid=(R,),
        in_specs=[
            pl.BlockSpec((R, NB), lambda r: (0, 0)),
            pl.BlockSpec((NB, H, H), lambda r: (0, 0, 0)),
            pl.BlockSpec((R + 1, H), lambda r: (0, 0)),
        ],
        out_specs=[
            pl.BlockSpec((1, HP, HP), lambda r: (r, 0, 0)),
            pl.BlockSpec((32, HP), lambda r: (0, 0)),
        ],
        out_shape=[
            jax.ShapeDtypeStruct((R, HP, HP), jnp.float32),
            jax.ShapeDtypeStruct((32, HP), jnp.float32),
        ],
    )(comp2, bases2, emb_rel)


def _hfix_body(part_ref, root_ref, bias_ref, out_ref):
    sp = part_ref[0] + part_ref[1]
    rb = root_ref[...] + bias_ref[...]
    rb = jnp.concatenate(
        [rb, jnp.zeros((rb.shape[0], HP - H), jnp.float32)], axis=1)
    out_ref[...] = jnp.maximum(sp + rb, 0.0)


def _hfix(hpart, root1, bias1):
    return pl.pallas_call(
        _hfix_body,
        grid=(10,),
        in_specs=[
            pl.BlockSpec((NC, N // 10, HP), lambda i: (0, i, 0)),
            pl.BlockSpec((N // 10, H), lambda i: (i, 0)),
            pl.BlockSpec((1, H), lambda i: (0, 0)),
        ],
        out_specs=pl.BlockSpec((N // 10, HP), lambda i: (i, 0)),
        out_shape=jax.ShapeDtypeStruct((N, HP), jnp.float32),
    )(hpart, root1, bias1.reshape(1, H))


_BNZ = 1000


def _z_body(h_ref, w_ref, out_ref):
    hh = h_ref[...]
    for r in range(R):
        out_ref[:, r, :] = jnp.dot(hh, w_ref[r],
                                   preferred_element_type=jnp.float32)


def _ztab(h_pad, w2m):
    out = pl.pallas_call(
        _z_body,
        grid=(N // _BNZ,),
        in_specs=[
            pl.BlockSpec((_BNZ, HP), lambda i: (i, 0)),
            pl.BlockSpec((R, HP, HP), lambda i: (0, 0, 0)),
        ],
        out_specs=pl.BlockSpec((_BNZ, R, HP), lambda i: (i, 0, 0)),
        out_shape=jax.ShapeDtypeStruct((N, R, HP), jnp.float32),
    )(h_pad, w2m)
    return out.reshape(N * R, HP)


def _final_body(part_ref, h_ref, root_ref, bias_ref, out_ref):
    o = part_ref[0] + part_ref[1]
    rp = jnp.concatenate(
        [root_ref[...], jnp.zeros((H, HP - H), jnp.float32)], axis=1)
    o = o + jnp.dot(h_ref[...][:, :H], rp,
                    preferred_element_type=jnp.float32)
    bp = jnp.concatenate(
        [bias_ref[...], jnp.zeros((1, HP - H), jnp.float32)], axis=1)
    out_ref[...] = (o + bp)[:, :H]


def _final(outpart, h_pad, root2, bias2):
    return pl.pallas_call(
        _final_body,
        grid=(10,),
        in_specs=[
            pl.BlockSpec((NC, N // 10, HP), lambda i: (0, i, 0)),
            pl.BlockSpec((N // 10, HP), lambda i: (i, 0)),
            pl.BlockSpec((H, H), lambda i: (0, 0)),
            pl.BlockSpec((1, H), lambda i: (0, 0)),
        ],
        out_specs=pl.BlockSpec((N // 10, H), lambda i: (i, 0)),
        out_shape=jax.ShapeDtypeStruct((N, H), jnp.float32),
    )(outpart, h_pad, root2, bias2.reshape(1, H))


def _relfix_body(in_ref, out_ref):
    out_ref[...] = in_ref[...][:, :H]


def _relfix(relp):
    return pl.pallas_call(
        _relfix_body,
        grid=(40,),
        in_specs=[pl.BlockSpec((E // 40, HP), lambda i: (i, 0))],
        out_specs=pl.BlockSpec((E // 40, H), lambda i: (i, 0)),
        out_shape=jax.ShapeDtypeStruct((E, H), jnp.float32),
    )(relp)


def kernel(edge_index, edge_type, bases1, comp1, root1, bias1,
           bases2, comp2, root2, bias2, emb_rel):
    cnt2, idx1, idx2, combs, dsts = _prep(
        edge_index[0], edge_index[1], edge_type)
    rcnt = _rcnt(cnt2.reshape(NC, NS * CPT)[:, :NR])
    w1tab = _w1p(comp1, bases1)
    w2m, tblz = _w2prep(comp2, bases2, emb_rel)

    hpart = _conv(w1tab, rcnt, idx1, combs, dsts).reshape(NC, NP, HP)[:, :N]
    h_pad = _hfix(hpart, root1, bias1)
    ztab = _ztab(h_pad, w2m)
    outpart = _conv(ztab, rcnt, idx2, combs, dsts).reshape(NC, NP, HP)[:, :N]
    out = _final(outpart, h_pad, root2, bias2)

    rel_embedded = _relfix(_rel(tblz, edge_type))
    return (out, rel_embedded)


# speedup vs baseline: 7.7211x; 2.2762x over previous
import jax
import jax.numpy as jnp
from jax import lax
from jax.experimental import pallas as pl
from jax.experimental.pallas import tpu as pltpu
from jax.experimental.pallas import tpu_sc as plsc

N = 10000
E = 320000
R = 16
H = 100
NB = 30
HP = 128
NP = 10240
NR = N * R
NC = 2
NS = 16
NW = NC * NS
C = 1280
CB = 128
NJ = C // CB
NJP = 16
NCHUNK = E // C
TPW = (NCHUNK + NW - 1) // NW
NPT = NP // NS
CPT = 10240

_mesh = plsc.VectorSubcoreMesh(
    core_axis_name="c", subcore_axis_name="s", num_cores=NC, num_subcores=NS)


def _worker_ids():
    c = lax.axis_index("c")
    s = lax.axis_index("s")
    return c, s, c * NS + s


def _prep_body(es_ref, ed_ref, et_ref,
               cnt2_ref, idx1_ref, idx2_ref, comb_ref, dst_ref,
               src_v, dst_v, rel_v, i1_v, i2_v, cb_v, d2_v, ones_v, zb_v,
               cnt_sh):
    c, s, wid = _worker_ids()
    for k in range(CB // 16):
        ones_v[pl.ds(k * 16, 16)] = jnp.ones((16,), jnp.float32)

    def zrow(k, carry):
        zb_v[pl.ds(k * 16, 16)] = jnp.zeros((16,), jnp.float32)
        return carry

    lax.fori_loop(0, 2048 // 16, zrow, 0)
    for j in range(CPT // 2048):
        pltpu.sync_copy(zb_v, cnt_sh.at[pl.ds(s * CPT + j * 2048, 2048)])
    plsc.subcore_barrier()

    def chunk(t, carry):
        i = wid + t * NW

        @pl.when(i < NCHUNK)
        def _():
            base = i * C
            pltpu.sync_copy(es_ref.at[pl.ds(base, C)], src_v)
            pltpu.sync_copy(ed_ref.at[pl.ds(base, C)], dst_v)
            pltpu.sync_copy(et_ref.at[pl.ds(base, C)], rel_v)
            for k in range(C // 16):
                row = k // (CB // 16)
                col = (k % (CB // 16)) * 16
                sl = pl.ds(k * 16, 16)
                sv = src_v[sl]
                dv = dst_v[sl]
                rv = rel_v[sl]
                cb_v[row, pl.ds(col, 16)] = dv * R + rv
                i1_v[row, pl.ds(col, 16)] = rv * N + sv
                i2_v[row, pl.ds(col, 16)] = sv * R + rv
                d2_v[row, pl.ds(col, 16)] = dv
            for j in range(NJ):
                pltpu.sync_copy(ones_v, cnt_sh.at[cb_v.at[j]], add=True)
            pltpu.sync_copy(i1_v, idx1_ref.at[i])
            pltpu.sync_copy(i2_v, idx2_ref.at[i])
            pltpu.sync_copy(cb_v, comb_ref.at[i])
            pltpu.sync_copy(d2_v, dst_ref.at[i])
        return carry

    lax.fori_loop(0, TPW, chunk, 0)
    plsc.subcore_barrier()
    pltpu.sync_copy(cnt_sh.at[pl.ds(s * CPT, CPT)], cnt2_ref.at[c, s])


_prep = pl.kernel(
    _prep_body,
    out_type=(
        jax.ShapeDtypeStruct((NC, NS, CPT), jnp.float32),
        jax.ShapeDtypeStruct((NCHUNK, NJP, CB), jnp.int32),
        jax.ShapeDtypeStruct((NCHUNK, NJP, CB), jnp.int32),
        jax.ShapeDtypeStruct((NCHUNK, NJP, CB), jnp.int32),
        jax.ShapeDtypeStruct((NCHUNK, NJP, CB), jnp.int32),
    ),
    mesh=_mesh,
    scratch_types=[
        pltpu.VMEM((C,), jnp.int32),
        pltpu.VMEM((C,), jnp.int32),
        pltpu.VMEM((C,), jnp.int32),
        pltpu.VMEM((NJP, CB), jnp.int32),
        pltpu.VMEM((NJP, CB), jnp.int32),
        pltpu.VMEM((NJP, CB), jnp.int32),
        pltpu.VMEM((NJP, CB), jnp.int32),
        pltpu.VMEM((CB,), jnp.float32),
        pltpu.VMEM((2048,), jnp.float32),
        pltpu.VMEM_SHARED((NS * CPT,), jnp.float32),
    ],
)


NBUF = 2


def _conv_body(tab_ref, rcnt_ref, idx_ref, comb_ref, dst_ref,
               part_ref,
               i_v, c_v, d_v, n_v, r0, r1, acc_sh,
               nsem, gsem, ssem):
    c, s, wid = _worker_ids()
    rows = [r0, r1]

    def zrow(r, carry):
        for q in range(HP // 16):
            r0[r, pl.ds(q * 16, 16)] = jnp.zeros((16,), jnp.float32)
        return carry

    lax.fori_loop(0, CB, zrow, 0)
    for j in range(NPT // CB):
        pltpu.sync_copy(r0, acc_sh.at[pl.ds(s * NPT + j * CB, CB)])
    plsc.subcore_barrier()

    def chunk(t, carry):
        i = wid + t * NW

        @pl.when(i < NCHUNK)
        def _():
            pltpu.sync_copy(idx_ref.at[i], i_v)
            pltpu.sync_copy(comb_ref.at[i], c_v)
            pltpu.sync_copy(dst_ref.at[i], d_v)
            for j in range(NJ):
                pltpu.async_copy(rcnt_ref.at[c_v.at[j]],
                                 n_v.at[pl.ds(j * CB, CB)], nsem)
            pltpu.async_copy(tab_ref.at[i_v.at[0]], rows[0], gsem.at[0])
            for j in range(NJ):
                pltpu.make_async_copy(rcnt_ref.at[c_v.at[j]],
                                      n_v.at[pl.ds(j * CB, CB)],
                                      nsem).wait()
            for j in range(NJ):
                b = j % NBUF
                o = (j + 1) % NBUF
                pltpu.make_async_copy(tab_ref.at[i_v.at[j]], rows[b],
                                      gsem.at[b]).wait()
                if j + 1 < NJ:
                    if j >= 1:
                        pltpu.make_async_copy(
                            rows[o], acc_sh.at[d_v.at[j - 1]],
                            ssem.at[o]).wait()
                    pltpu.async_copy(tab_ref.at[i_v.at[j + 1]], rows[o],
                                     gsem.at[o])

                def scale(g, carry2):
                    e = g * 2
                    for u in range(2):
                        nbv = n_v[pl.ds((e + u) + j * CB, 16)]
                        nb = jnp.full((16,), nbv[0], jnp.float32)
                        for q in range(HP // 16):
                            sl = pl.ds(q * 16, 16)
                            rows[b][e + u, sl] = rows[b][e + u, sl] * nb
                    return carry2

                lax.fori_loop(0, CB // 2, scale, 0, unroll=2)
                pltpu.async_copy(rows[b], acc_sh.at[d_v.at[j]],
                                 ssem.at[b], add=True)
            for j in range(NJ - NBUF, NJ):
                b = j % NBUF
                pltpu.make_async_copy(rows[b], acc_sh.at[d_v.at[j]],
                                      ssem.at[b]).wait()
        return carry

    lax.fori_loop(0, TPW, chunk, 0)
    plsc.subcore_barrier()
    pltpu.sync_copy(acc_sh.at[pl.ds(s * NPT, NPT)], part_ref.at[c, s])


_conv = pl.kernel(
    _conv_body,
    out_type=jax.ShapeDtypeStruct((NC, NS, NPT, HP), jnp.float32),
    mesh=_mesh,
    scratch_types=[
        pltpu.VMEM((NJP, CB), jnp.int32),
        pltpu.VMEM((NJP, CB), jnp.int32),
        pltpu.VMEM((NJP, CB), jnp.int32),
        pltpu.VMEM((NJP * CB,), jnp.float32),
        pltpu.VMEM((CB, HP), jnp.float32),
        pltpu.VMEM((CB, HP), jnp.float32),
        pltpu.VMEM_SHARED((NP, HP), jnp.float32),
        pltpu.SemaphoreType.DMA,
        pltpu.SemaphoreType.DMA((NBUF,)),
        pltpu.SemaphoreType.DMA((NBUF,)),
    ],
)


_BE = 4000


def _reltc_body(et_ref, tbl_ref, out_ref):
    et = et_ref[...]
    onehot = (et == lax.broadcasted_iota(jnp.int32, (1, 32), 1)
              ).astype(jnp.float32)
    res = jnp.dot(onehot, tbl_ref[...], preferred_element_type=jnp.float32)
    out_ref[...] = res[:, :H]


def _reltc(edge_type, tblz):
    return pl.pallas_call(
        _reltc_body,
        grid=(E // _BE,),
        in_specs=[
            pl.BlockSpec((_BE, 1), lambda i: (i, 0)),
            pl.BlockSpec((32, HP), lambda i: (0, 0)),
        ],
        out_specs=pl.BlockSpec((_BE, H), lambda i: (i, 0)),
        out_shape=jax.ShapeDtypeStruct((E, H), jnp.float32),
    )(edge_type.reshape(E, 1), tblz)


def _rcnt_body(cnt_ref, out_ref):
    out_ref[...] = 1.0 / jnp.maximum(cnt_ref[0] + cnt_ref[1], 1.0)


def _rcnt(cnt2):
    cnt3 = cnt2.reshape(NC, NR // 128, 128)
    out = pl.pallas_call(
        _rcnt_body,
        grid=(1,),
        in_specs=[pl.BlockSpec((NC, NR // 128, 128), lambda i: (0, 0, 0))],
        out_specs=pl.BlockSpec((NR // 128, 128), lambda i: (0, 0)),
        out_shape=jax.ShapeDtypeStruct((NR // 128, 128), jnp.float32),
    )(cnt3)
    return out.reshape(NR)


_BN1 = 1000


def _w1p_body(comp_ref, bases_ref, out_ref):
    b = bases_ref[...].reshape(NB, _BN1 * H)
    m = jnp.dot(comp_ref[...], b,
                preferred_element_type=jnp.float32).reshape(R, _BN1, H)
    pad = jnp.zeros((R, _BN1, HP - H), jnp.float32)
    out_ref[...] = jnp.concatenate([m, pad], axis=2)


def _w1p(comp1, bases1):
    out = pl.pallas_call(
        _w1p_body,
        grid=(N // _BN1,),
        in_specs=[
            pl.BlockSpec((R, NB), lambda i: (0, 0)),
            pl.BlockSpec((NB, _BN1, H), lambda i: (0, i, 0)),
        ],
        out_specs=pl.BlockSpec((R, _BN1, HP), lambda i: (0, i, 0)),
        out_shape=jax.ShapeDtypeStruct((R, N, HP), jnp.float32),
    )(comp1, bases1)
    return out.reshape(R * N, HP)


def _w2prep_body(comp_ref, bases_ref, emb_ref, w2m_ref, tbl_ref):
    sel = (lax.broadcasted_iota(jnp.int32, (1, R), 1)
           == pl.program_id(0)).astype(jnp.float32)
    row = jnp.dot(sel, comp_ref[...], preferred_element_type=jnp.float32)
    v = jnp.zeros((H, H), jnp.float32)
    for b in range(NB):
        v = v + row[0, b] * bases_ref[b]
    v = jnp.concatenate([v, jnp.zeros((H, HP - H), jnp.float32)], axis=1)
    v = jnp.concatenate([v, jnp.zeros((HP - H, HP), jnp.float32)], axis=0)
    w2m_ref[...] = v[None]
    e = emb_ref[...]
    ep = jnp.concatenate(
        [jnp.zeros((1, H), jnp.float32), e[1:]], axis=0)
    ep = jnp.concatenate(
        [ep, jnp.zeros((R + 1, HP - H), jnp.float32)], axis=1)
    tbl_ref[...] = jnp.concatenate(
        [ep, jnp.zeros((32 - (R + 1), HP), jnp.float32)], axis=0)


def _w2prep(comp2, bases2, emb_rel):
    return pl.pallas_call(
        _w2prep_body,
        grid=(R,),
        in_specs=[
            pl.BlockSpec((R, NB), lambda r: (0, 0)),
            pl.BlockSpec((NB, H, H), lambda r: (0, 0, 0)),
            pl.BlockSpec((R + 1, H), lambda r: (0, 0)),
        ],
        out_specs=[
            pl.BlockSpec((1, HP, HP), lambda r: (r, 0, 0)),
            pl.BlockSpec((32, HP), lambda r: (0, 0)),
        ],
        out_shape=[
            jax.ShapeDtypeStruct((R, HP, HP), jnp.float32),
            jax.ShapeDtypeStruct((32, HP), jnp.float32),
        ],
    )(comp2, bases2, emb_rel)


def _hfix_body(part_ref, root_ref, bias_ref, out_ref):
    sp = part_ref[0] + part_ref[1]
    rb = root_ref[...] + bias_ref[...]
    rb = jnp.concatenate(
        [rb, jnp.zeros((rb.shape[0], HP - H), jnp.float32)], axis=1)
    out_ref[...] = jnp.maximum(sp + rb, 0.0)


def _hfix(hpart, root1, bias1):
    return pl.pallas_call(
        _hfix_body,
        grid=(10,),
        in_specs=[
            pl.BlockSpec((NC, N // 10, HP), lambda i: (0, i, 0)),
            pl.BlockSpec((N // 10, H), lambda i: (i, 0)),
            pl.BlockSpec((1, H), lambda i: (0, 0)),
        ],
        out_specs=pl.BlockSpec((N // 10, HP), lambda i: (i, 0)),
        out_shape=jax.ShapeDtypeStruct((N, HP), jnp.float32),
    )(hpart, root1, bias1.reshape(1, H))


_BNZ = 1000


def _z_body(h_ref, w_ref, out_ref):
    hh = h_ref[...]
    for r in range(R):
        out_ref[:, r, :] = jnp.dot(hh, w_ref[r],
                                   preferred_element_type=jnp.float32)


def _ztab(h_pad, w2m):
    out = pl.pallas_call(
        _z_body,
        grid=(N // _BNZ,),
        in_specs=[
            pl.BlockSpec((_BNZ, HP), lambda i: (i, 0)),
            pl.BlockSpec((R, HP, HP), lambda i: (0, 0, 0)),
        ],
        out_specs=pl.BlockSpec((_BNZ, R, HP), lambda i: (i, 0, 0)),
        out_shape=jax.ShapeDtypeStruct((N, R, HP), jnp.float32),
    )(h_pad, w2m)
    return out.reshape(N * R, HP)


def _final_body(part_ref, h_ref, root_ref, bias_ref, out_ref):
    o = part_ref[0] + part_ref[1]
    rp = jnp.concatenate(
        [root_ref[...], jnp.zeros((H, HP - H), jnp.float32)], axis=1)
    o = o + jnp.dot(h_ref[...][:, :H], rp,
                    preferred_element_type=jnp.float32)
    bp = jnp.concatenate(
        [bias_ref[...], jnp.zeros((1, HP - H), jnp.float32)], axis=1)
    out_ref[...] = (o + bp)[:, :H]


def _final(outpart, h_pad, root2, bias2):
    return pl.pallas_call(
        _final_body,
        grid=(10,),
        in_specs=[
            pl.BlockSpec((NC, N // 10, HP), lambda i: (0, i, 0)),
            pl.BlockSpec((N // 10, HP), lambda i: (i, 0)),
            pl.BlockSpec((H, H), lambda i: (0, 0)),
            pl.BlockSpec((1, H), lambda i: (0, 0)),
        ],
        out_specs=pl.BlockSpec((N // 10, H), lambda i: (i, 0)),
        out_shape=jax.ShapeDtypeStruct((N, H), jnp.float32),
    )(outpart, h_pad, root2, bias2.reshape(1, H))


def kernel(edge_index, edge_type, bases1, comp1, root1, bias1,
           bases2, comp2, root2, bias2, emb_rel):
    cnt2, idx1, idx2, combs, dsts = _prep(
        edge_index[0], edge_index[1], edge_type)
    rcnt = _rcnt(cnt2.reshape(NC, NS * CPT)[:, :NR])
    w1tab = _w1p(comp1, bases1)
    w2m, tblz = _w2prep(comp2, bases2, emb_rel)

    hpart = _conv(w1tab, rcnt, idx1, combs, dsts).reshape(NC, NP, HP)[:, :N]
    h_pad = _hfix(hpart, root1, bias1)
    ztab = _ztab(h_pad, w2m)
    outpart = _conv(ztab, rcnt, idx2, combs, dsts).reshape(NC, NP, HP)[:, :N]
    out = _final(outpart, h_pad, root2, bias2)

    rel_embedded = _reltc(edge_type, tblz)
    return (out, rel_embedded)


# speedup vs baseline: 7.8223x; 1.0131x over previous
import jax
import jax.numpy as jnp
from jax import lax
from jax.experimental import pallas as pl
from jax.experimental.pallas import tpu as pltpu
from jax.experimental.pallas import tpu_sc as plsc

N = 10000
E = 320000
R = 16
H = 100
NB = 30
HP = 128
NP = 10240
NR = N * R
NC = 2
NS = 16
NW = NC * NS
C = 1280
CB = 128
NJ = C // CB
NJP = 16
NCHUNK = E // C
TPW = (NCHUNK + NW - 1) // NW
NPT = NP // NS
CPT = 10240

_mesh = plsc.VectorSubcoreMesh(
    core_axis_name="c", subcore_axis_name="s", num_cores=NC, num_subcores=NS)


def _worker_ids():
    c = lax.axis_index("c")
    s = lax.axis_index("s")
    return c, s, c * NS + s


def _prep_body(es_ref, ed_ref, et_ref,
               cnt2_ref, idx1_ref, idx2_ref, comb_ref, dst_ref,
               src_v, dst_v, rel_v, i1_v, i2_v, cb_v, d2_v, ones_v, zb_v,
               cnt_sh):
    c, s, wid = _worker_ids()
    for k in range(CB // 16):
        ones_v[pl.ds(k * 16, 16)] = jnp.ones((16,), jnp.float32)

    def zrow(k, carry):
        zb_v[pl.ds(k * 16, 16)] = jnp.zeros((16,), jnp.float32)
        return carry

    lax.fori_loop(0, 2048 // 16, zrow, 0)
    for j in range(CPT // 2048):
        pltpu.sync_copy(zb_v, cnt_sh.at[pl.ds(s * CPT + j * 2048, 2048)])
    plsc.subcore_barrier()

    def chunk(t, carry):
        i = wid + t * NW

        @pl.when(i < NCHUNK)
        def _():
            base = i * C
            pltpu.sync_copy(es_ref.at[pl.ds(base, C)], src_v)
            pltpu.sync_copy(ed_ref.at[pl.ds(base, C)], dst_v)
            pltpu.sync_copy(et_ref.at[pl.ds(base, C)], rel_v)
            for k in range(C // 16):
                row = k // (CB // 16)
                col = (k % (CB // 16)) * 16
                sl = pl.ds(k * 16, 16)
                sv = src_v[sl]
                dv = dst_v[sl]
                rv = rel_v[sl]
                cb_v[row, pl.ds(col, 16)] = dv * R + rv
                i1_v[row, pl.ds(col, 16)] = rv * N + sv
                i2_v[row, pl.ds(col, 16)] = sv * R + rv
                d2_v[row, pl.ds(col, 16)] = dv
            for j in range(NJ):
                pltpu.sync_copy(ones_v, cnt_sh.at[cb_v.at[j]], add=True)
            pltpu.sync_copy(i1_v, idx1_ref.at[i])
            pltpu.sync_copy(i2_v, idx2_ref.at[i])
            pltpu.sync_copy(cb_v, comb_ref.at[i])
            pltpu.sync_copy(d2_v, dst_ref.at[i])
        return carry

    lax.fori_loop(0, TPW, chunk, 0)
    plsc.subcore_barrier()
    pltpu.sync_copy(cnt_sh.at[pl.ds(s * CPT, CPT)], cnt2_ref.at[c, s])


_prep = pl.kernel(
    _prep_body,
    out_type=(
        jax.ShapeDtypeStruct((NC, NS, CPT), jnp.float32),
        jax.ShapeDtypeStruct((NCHUNK, NJP, CB), jnp.int32),
        jax.ShapeDtypeStruct((NCHUNK, NJP, CB), jnp.int32),
        jax.ShapeDtypeStruct((NCHUNK, NJP, CB), jnp.int32),
        jax.ShapeDtypeStruct((NCHUNK, NJP, CB), jnp.int32),
    ),
    mesh=_mesh,
    scratch_types=[
        pltpu.VMEM((C,), jnp.int32),
        pltpu.VMEM((C,), jnp.int32),
        pltpu.VMEM((C,), jnp.int32),
        pltpu.VMEM((NJP, CB), jnp.int32),
        pltpu.VMEM((NJP, CB), jnp.int32),
        pltpu.VMEM((NJP, CB), jnp.int32),
        pltpu.VMEM((NJP, CB), jnp.int32),
        pltpu.VMEM((CB,), jnp.float32),
        pltpu.VMEM((2048,), jnp.float32),
        pltpu.VMEM_SHARED((NS * CPT,), jnp.float32),
    ],
)


NBUF = 2


def _conv_body(tab_ref, rcnt_ref, idx_ref, comb_ref, dst_ref,
               part_ref,
               i_v, c_v, d_v, n_v, r0, r1, acc_sh,
               nsem, gsem, ssem):
    c, s, wid = _worker_ids()
    rows = [r0, r1]

    def zrow(r, carry):
        for q in range(HP // 16):
            r0[r, pl.ds(q * 16, 16)] = jnp.zeros((16,), jnp.float32)
        return carry

    lax.fori_loop(0, CB, zrow, 0)
    for j in range(NPT // CB):
        pltpu.sync_copy(r0, acc_sh.at[pl.ds(s * NPT + j * CB, CB)])
    plsc.subcore_barrier()

    def chunk(t, carry):
        i = wid + t * NW

        @pl.when(i < NCHUNK)
        def _():
            pltpu.sync_copy(idx_ref.at[i], i_v)
            pltpu.sync_copy(comb_ref.at[i], c_v)
            pltpu.sync_copy(dst_ref.at[i], d_v)
            for j in range(NJ):
                pltpu.async_copy(rcnt_ref.at[c_v.at[j]],
                                 n_v.at[pl.ds(j * CB, CB)], nsem)
            pltpu.async_copy(tab_ref.at[i_v.at[0]], rows[0], gsem.at[0])
            for j in range(NJ):
                pltpu.make_async_copy(rcnt_ref.at[c_v.at[j]],
                                      n_v.at[pl.ds(j * CB, CB)],
                                      nsem).wait()
            for j in range(NJ):
                b = j % NBUF
                o = (j + 1) % NBUF
                pltpu.make_async_copy(tab_ref.at[i_v.at[j]], rows[b],
                                      gsem.at[b]).wait()
                if j + 1 < NJ:
                    if j >= 1:
                        pltpu.make_async_copy(
                            rows[o], acc_sh.at[d_v.at[j - 1]],
                            ssem.at[o]).wait()
                    pltpu.async_copy(tab_ref.at[i_v.at[j + 1]], rows[o],
                                     gsem.at[o])

                def scale(g, carry2):
                    e = g * 2
                    for u in range(2):
                        nbv = n_v[pl.ds((e + u) + j * CB, 16)]
                        nb = jnp.full((16,), nbv[0], jnp.float32)
                        for q in range(HP // 16):
                            sl = pl.ds(q * 16, 16)
                            rows[b][e + u, sl] = rows[b][e + u, sl] * nb
                    return carry2

                lax.fori_loop(0, CB // 2, scale, 0, unroll=4)
                pltpu.async_copy(rows[b], acc_sh.at[d_v.at[j]],
                                 ssem.at[b], add=True)
            for j in range(NJ - NBUF, NJ):
                b = j % NBUF
                pltpu.make_async_copy(rows[b], acc_sh.at[d_v.at[j]],
                                      ssem.at[b]).wait()
        return carry

    lax.fori_loop(0, TPW, chunk, 0)
    plsc.subcore_barrier()
    pltpu.sync_copy(acc_sh.at[pl.ds(s * NPT, NPT)], part_ref.at[c, s])


_conv = pl.kernel(
    _conv_body,
    out_type=jax.ShapeDtypeStruct((NC, NS, NPT, HP), jnp.float32),
    mesh=_mesh,
    scratch_types=[
        pltpu.VMEM((NJP, CB), jnp.int32),
        pltpu.VMEM((NJP, CB), jnp.int32),
        pltpu.VMEM((NJP, CB), jnp.int32),
        pltpu.VMEM((NJP * CB,), jnp.float32),
        pltpu.VMEM((CB, HP), jnp.float32),
        pltpu.VMEM((CB, HP), jnp.float32),
        pltpu.VMEM_SHARED((NP, HP), jnp.float32),
        pltpu.SemaphoreType.DMA,
        pltpu.SemaphoreType.DMA((NBUF,)),
        pltpu.SemaphoreType.DMA((NBUF,)),
    ],
)


_BE = 4000


def _reltc_body(et_ref, tbl_ref, out_ref):
    et = et_ref[...]
    onehot = (et == lax.broadcasted_iota(jnp.int32, (1, 32), 1)
              ).astype(jnp.float32)
    res = jnp.dot(onehot, tbl_ref[...], preferred_element_type=jnp.float32)
    out_ref[...] = res[:, :H]


def _reltc(edge_type, tblz):
    return pl.pallas_call(
        _reltc_body,
        grid=(E // _BE,),
        in_specs=[
            pl.BlockSpec((_BE, 1), lambda i: (i, 0)),
            pl.BlockSpec((32, HP), lambda i: (0, 0)),
        ],
        out_specs=pl.BlockSpec((_BE, H), lambda i: (i, 0)),
        out_shape=jax.ShapeDtypeStruct((E, H), jnp.float32),
    )(edge_type.reshape(E, 1), tblz)


def _rcnt_body(cnt_ref, out_ref):
    out_ref[...] = 1.0 / jnp.maximum(cnt_ref[0] + cnt_ref[1], 1.0)


def _rcnt(cnt2):
    nrp = NS * CPT
    cnt3 = cnt2.reshape(NC, nrp // 128, 128)
    out = pl.pallas_call(
        _rcnt_body,
        grid=(1,),
        in_specs=[pl.BlockSpec((NC, nrp // 128, 128), lambda i: (0, 0, 0))],
        out_specs=pl.BlockSpec((nrp // 128, 128), lambda i: (0, 0)),
        out_shape=jax.ShapeDtypeStruct((nrp // 128, 128), jnp.float32),
    )(cnt3)
    return out.reshape(nrp)


_BN1 = 1000


def _w1p_body(comp_ref, bases_ref, out_ref):
    b = bases_ref[...].reshape(NB, _BN1 * H)
    m = jnp.dot(comp_ref[...], b,
                preferred_element_type=jnp.float32).reshape(R, _BN1, H)
    pad = jnp.zeros((R, _BN1, HP - H), jnp.float32)
    out_ref[...] = jnp.concatenate([m, pad], axis=2)


def _w1p(comp1, bases1):
    out = pl.pallas_call(
        _w1p_body,
        grid=(N // _BN1,),
        in_specs=[
            pl.BlockSpec((R, NB), lambda i: (0, 0)),
            pl.BlockSpec((NB, _BN1, H), lambda i: (0, i, 0)),
        ],
        out_specs=pl.BlockSpec((R, _BN1, HP), lambda i: (0, i, 0)),
        out_shape=jax.ShapeDtypeStruct((R, N, HP), jnp.float32),
    )(comp1, bases1)
    return out.reshape(R * N, HP)


def _w2prep_body(comp_ref, bases_ref, emb_ref, w2m_ref, tbl_ref):
    sel = (lax.broadcasted_iota(jnp.int32, (1, R), 1)
           == pl.program_id(0)).astype(jnp.float32)
    row = jnp.dot(sel, comp_ref[...], preferred_element_type=jnp.float32)
    v = jnp.zeros((H, H), jnp.float32)
    for b in range(NB):
        v = v + row[0, b] * bases_ref[b]
    v = jnp.concatenate([v, jnp.zeros((H, HP - H), jnp.float32)], axis=1)
    v = jnp.concatenate([v, jnp.zeros((HP - H, HP), jnp.float32)], axis=0)
    w2m_ref[...] = v[None]
    e = emb_ref[...]
    ep = jnp.concatenate(
        [jnp.zeros((1, H), jnp.float32), e[1:]], axis=0)
    ep = jnp.concatenate(
        [ep, jnp.zeros((R + 1, HP - H), jnp.float32)], axis=1)
    tbl_ref[...] = jnp.concatenate(
        [ep, jnp.zeros((32 - (R + 1), HP), jnp.float32)], axis=0)


def _w2prep(comp2, bases2, emb_rel):
    return pl.pallas_call(
        _w2prep_body,
        grid=(R,),
        in_specs=[
            pl.BlockSpec((R, NB), lambda r: (0, 0)),
            pl.BlockSpec((NB, H, H), lambda r: (0, 0, 0)),
            pl.BlockSpec((R + 1, H), lambda r: (0, 0)),
        ],
        out_specs=[
            pl.BlockSpec((1, HP, HP), lambda r: (r, 0, 0)),
            pl.BlockSpec((32, HP), lambda r: (0, 0)),
        ],
        out_shape=[
            jax.ShapeDtypeStruct((R, HP, HP), jnp.float32),
            jax.ShapeDtypeStruct((32, HP), jnp.float32),
        ],
    )(comp2, bases2, emb_rel)


def _hfix_body(part_ref, root_ref, bias_ref, out_ref):
    sp = part_ref[0] + part_ref[1]
    rb = root_ref[...] + bias_ref[...]
    rb = jnp.concatenate(
        [rb, jnp.zeros((rb.shape[0], HP - H), jnp.float32)], axis=1)
    out_ref[...] = jnp.maximum(sp + rb, 0.0)


def _hfix(hpart, root1, bias1):
    return pl.pallas_call(
        _hfix_body,
        grid=(10,),
        in_specs=[
            pl.BlockSpec((NC, N // 10, HP), lambda i: (0, i, 0)),
            pl.BlockSpec((N // 10, H), lambda i: (i, 0)),
            pl.BlockSpec((1, H), lambda i: (0, 0)),
        ],
        out_specs=pl.BlockSpec((N // 10, HP), lambda i: (i, 0)),
        out_shape=jax.ShapeDtypeStruct((N, HP), jnp.float32),
    )(hpart, root1, bias1.reshape(1, H))


_BNZ = 1000


def _z_body(h_ref, w_ref, out_ref):
    hh = h_ref[...]
    for r in range(R):
        out_ref[:, r, :] = jnp.dot(hh, w_ref[r],
                                   preferred_element_type=jnp.float32)


def _ztab(h_pad, w2m):
    out = pl.pallas_call(
        _z_body,
        grid=(N // _BNZ,),
        in_specs=[
            pl.BlockSpec((_BNZ, HP), lambda i: (i, 0)),
            pl.BlockSpec((R, HP, HP), lambda i: (0, 0, 0)),
        ],
        out_specs=pl.BlockSpec((_BNZ, R, HP), lambda i: (i, 0, 0)),
        out_shape=jax.ShapeDtypeStruct((N, R, HP), jnp.float32),
    )(h_pad, w2m)
    return out.reshape(N * R, HP)


def _final_body(part_ref, h_ref, root_ref, bias_ref, out_ref):
    o = part_ref[0] + part_ref[1]
    rp = jnp.concatenate(
        [root_ref[...], jnp.zeros((H, HP - H), jnp.float32)], axis=1)
    o = o + jnp.dot(h_ref[...][:, :H], rp,
                    preferred_element_type=jnp.float32)
    bp = jnp.concatenate(
        [bias_ref[...], jnp.zeros((1, HP - H), jnp.float32)], axis=1)
    out_ref[...] = (o + bp)[:, :H]


def _final(outpart, h_pad, root2, bias2):
    return pl.pallas_call(
        _final_body,
        grid=(10,),
        in_specs=[
            pl.BlockSpec((NC, N // 10, HP), lambda i: (0, i, 0)),
            pl.BlockSpec((N // 10, HP), lambda i: (i, 0)),
            pl.BlockSpec((H, H), lambda i: (0, 0)),
            pl.BlockSpec((1, H), lambda i: (0, 0)),
        ],
        out_specs=pl.BlockSpec((N // 10, H), lambda i: (i, 0)),
        out_shape=jax.ShapeDtypeStruct((N, H), jnp.float32),
    )(outpart, h_pad, root2, bias2.reshape(1, H))


def kernel(edge_index, edge_type, bases1, comp1, root1, bias1,
           bases2, comp2, root2, bias2, emb_rel):
    w1tab = _w1p(comp1, bases1)
    w2m, tblz = _w2prep(comp2, bases2, emb_rel)
    cnt2, idx1, idx2, combs, dsts = _prep(
        edge_index[0], edge_index[1], edge_type)
    rcnt = _rcnt(cnt2.reshape(NC, NS * CPT))

    hpart = _conv(w1tab, rcnt, idx1, combs, dsts).reshape(NC, NP, HP)
    h_pad = _hfix(hpart, root1, bias1)
    ztab = _ztab(h_pad, w2m)
    outpart = _conv(ztab, rcnt, idx2, combs, dsts).reshape(NC, NP, HP)
    rel_embedded = _reltc(edge_type, tblz)
    out = _final(outpart, h_pad, root2, bias2)
    return (out, rel_embedded)
